# Initial kernel scaffold; baseline (speedup 1.0000x reference)
import functools

import jax
import jax.numpy as jnp
from jax import lax
from jax.experimental import pallas as pl
from jax.experimental.pallas import tpu as pltpu
from jax.experimental.pallas import tpu_sc as plsc

VOCAB = 100000
D = 128
B = 16384
NC = 2
NS = 16
NW = NC * NS
BPW = B // NW
CHUNK = 128
NCHUNK = BPW // CHUNK
VPR = D // 16
NTAB = 4


def _sc_body(widx_hbm, t0, t1, t2, t3, mix_hbm, out_hbm,
             idx_v, rows0, rows1, mix_v, outbuf, sem0, sem1, semx):
    wid = lax.axis_index("s") * NC + lax.axis_index("c")
    base = wid * BPW
    pltpu.sync_copy(widx_hbm.at[pl.ds(base, BPW)], idx_v)

    mix_handles = [
        pltpu.async_copy(
            mix_hbm.at[idx_v.at[pl.ds(ck * CHUNK, CHUNK)]],
            mix_v.at[pl.ds(ck * CHUNK, CHUNK)],
            semx,
        )
        for ck in range(NCHUNK)
    ]

    tables = [t0, t1, t2, t3]
    bufs = [rows0, rows1]
    sems = [sem0, sem1]
    jobs = [(t, ck) for t in tables for ck in range(NCHUNK)]
    handles = [None, None]

    def issue(j):
        t, ck = jobs[j]
        b = j % 2
        handles[b] = pltpu.async_copy(
            t.at[idx_v.at[pl.ds(ck * CHUNK, CHUNK)]], bufs[b], sems[b])

    issue(0)
    issue(1)

    accs = tuple(jnp.zeros((16,), jnp.float32) for _ in range(VPR))

    def reduce_chunk(buf_ref, accs):
        def body(r, a):
            return tuple(a[c] + buf_ref[r, pl.ds(c * 16, 16)]
                         for c in range(VPR))
        return lax.fori_loop(0, CHUNK, body, accs)

    for j in range(len(jobs)):
        b = j % 2
        handles[b].wait()
        accs = reduce_chunk(bufs[b], accs)
        if j + 2 < len(jobs):
            issue(j + 2)

    tot = accs[0]
    for c in range(1, VPR):
        tot = tot + accs[c]

    for h in mix_handles:
        h.wait()
    iota = lax.iota(jnp.int32, 16)
    zeros = jnp.zeros((16,), jnp.int32)
    ones = jnp.ones((16,), jnp.int32)

    def mbody(i, m):
        rows = i * 16 + iota
        v0 = plsc.load_gather(mix_v, [rows, zeros])
        v1 = plsc.load_gather(mix_v, [rows, ones])
        return m + v0 + v1

    tot = tot + lax.fori_loop(0, BPW // 16, mbody,
                              jnp.zeros((16,), jnp.float32))

    outbuf[...] = tot
    pltpu.sync_copy(outbuf, out_hbm.at[wid])


@jax.jit
def _run(word_idxs, t0, t1, t2, t3, mixture):
    mesh = plsc.VectorSubcoreMesh(core_axis_name="c", subcore_axis_name="s")
    f = pl.kernel(
        _sc_body,
        out_type=jax.ShapeDtypeStruct((NW, 16), jnp.float32),
        mesh=mesh,
        scratch_types=[
            pltpu.VMEM((BPW,), jnp.int32),
            pltpu.VMEM((CHUNK, D), jnp.float32),
            pltpu.VMEM((CHUNK, D), jnp.float32),
            pltpu.VMEM((BPW, 2), jnp.float32),
            pltpu.VMEM((16,), jnp.float32),
            pltpu.SemaphoreType.DMA,
            pltpu.SemaphoreType.DMA,
            pltpu.SemaphoreType.DMA,
        ],
    )
    partials = f(word_idxs, t0, t1, t2, t3, mixture)
    return jnp.sum(partials)


def kernel(word_idxs, pos_idxs, neg_idxs, mus, logsigmas, mixture,
           mus_out, logsigmas_out):
    del pos_idxs, neg_idxs
    idx = word_idxs.astype(jnp.int32)
    t0 = mus.reshape(VOCAB, D)
    t1 = logsigmas.reshape(VOCAB, D)
    t2 = mus_out.reshape(VOCAB, D)
    t3 = logsigmas_out.reshape(VOCAB, D)
    return _run(idx, t0, t1, t2, t3, mixture)


# speedup vs baseline: 1.2603x; 1.2603x over previous
import functools

import jax
import jax.numpy as jnp
from jax import lax
from jax.experimental import pallas as pl
from jax.experimental.pallas import tpu as pltpu
from jax.experimental.pallas import tpu_sc as plsc

VOCAB = 100000
D = 128
B = 16384
NC = 2
NS = 16
NW = NC * NS
BPW = B // NW
CHUNK = 128
NCHUNK = BPW // CHUNK
VPR = D // 16
NTAB = 4


def _sc_body(widx_hbm, t0, t1, t2, t3, mix_hbm, out_hbm,
             idx_v, rows0, rows1, midx_v, mix_v, outbuf, sem0, sem1, semx):
    wid = lax.axis_index("s") * NC + lax.axis_index("c")
    base = wid * BPW
    pltpu.sync_copy(widx_hbm.at[pl.ds(base, BPW)], idx_v)

    def build_mix_idx(i, _):
        v = idx_v[pl.ds(i * 16, 16)]
        v2 = v + v
        midx_v[pl.ds(i * 16, 16)] = v2
        midx_v[pl.ds(BPW + i * 16, 16)] = v2 + 1
        return 0

    lax.fori_loop(0, BPW // 16, build_mix_idx, 0)
    mix_handles = [
        pltpu.async_copy(
            mix_hbm.at[midx_v.at[pl.ds(ck * CHUNK, CHUNK)]],
            mix_v.at[pl.ds(ck * CHUNK, CHUNK)],
            semx,
        )
        for ck in range(2 * NCHUNK)
    ]

    tables = [t0, t1, t2, t3]
    bufs = [rows0, rows1]
    sems = [sem0, sem1]
    jobs = [(t, ck) for t in tables for ck in range(NCHUNK)]
    handles = [None, None]

    def issue(j):
        t, ck = jobs[j]
        b = j % 2
        handles[b] = pltpu.async_copy(
            t.at[idx_v.at[pl.ds(ck * CHUNK, CHUNK)]], bufs[b], sems[b])

    issue(0)
    issue(1)

    accs = tuple(jnp.zeros((16,), jnp.float32) for _ in range(VPR))

    def reduce_chunk(buf_ref, accs):
        def body(r, a):
            return tuple(a[c] + buf_ref[r, pl.ds(c * 16, 16)]
                         for c in range(VPR))
        return lax.fori_loop(0, CHUNK, body, accs)

    for j in range(len(jobs)):
        b = j % 2
        handles[b].wait()
        accs = reduce_chunk(bufs[b], accs)
        if j + 2 < len(jobs):
            issue(j + 2)

    tot = accs[0]
    for c in range(1, VPR):
        tot = tot + accs[c]

    for h in mix_handles:
        h.wait()

    def mbody(i, m):
        return m + mix_v[pl.ds(i * 16, 16)]

    tot = tot + lax.fori_loop(0, 2 * BPW // 16, mbody,
                              jnp.zeros((16,), jnp.float32))

    outbuf[...] = tot
    pltpu.sync_copy(outbuf, out_hbm.at[wid])


@jax.jit
def _run(word_idxs, t0, t1, t2, t3, mixture):
    mesh = plsc.VectorSubcoreMesh(core_axis_name="c", subcore_axis_name="s")
    f = pl.kernel(
        _sc_body,
        out_type=jax.ShapeDtypeStruct((NW, 16), jnp.float32),
        mesh=mesh,
        scratch_types=[
            pltpu.VMEM((BPW,), jnp.int32),
            pltpu.VMEM((CHUNK, D), jnp.float32),
            pltpu.VMEM((CHUNK, D), jnp.float32),
            pltpu.VMEM((2 * BPW,), jnp.int32),
            pltpu.VMEM((2 * BPW,), jnp.float32),
            pltpu.VMEM((16,), jnp.float32),
            pltpu.SemaphoreType.DMA,
            pltpu.SemaphoreType.DMA,
            pltpu.SemaphoreType.DMA,
        ],
    )
    partials = f(word_idxs, t0, t1, t2, t3, mixture.reshape(2 * VOCAB))
    return jnp.sum(partials)


def kernel(word_idxs, pos_idxs, neg_idxs, mus, logsigmas, mixture,
           mus_out, logsigmas_out):
    del pos_idxs, neg_idxs
    idx = word_idxs.astype(jnp.int32)
    t0 = mus.reshape(VOCAB, D)
    t1 = logsigmas.reshape(VOCAB, D)
    t2 = mus_out.reshape(VOCAB, D)
    t3 = logsigmas_out.reshape(VOCAB, D)
    return _run(idx, t0, t1, t2, t3, mixture)


# speedup vs baseline: 4.0614x; 3.2224x over previous
import jax
import jax.numpy as jnp
from jax import lax
from jax.experimental import pallas as pl
from jax.experimental.pallas import tpu as pltpu
from jax.experimental.pallas import tpu_sc as plsc

VOCAB = 100000
NCOMP = 2
EMBED = 64
B = 16384
NC = 2
NS = 16
NW = NC * NS
BPW = B // NW
CHUNK = 128
NCHUNK = BPW // CHUNK

VCH = 2048
NBLK = (VOCAB + VCH - 1) // VCH


def _plane_sum_body(t0, t1, t2, t3, mix, out):
    i = pl.program_id(0)
    s = (t0[...].reshape(NCOMP * EMBED, VCH).sum(axis=0)
         + t1[...].reshape(NCOMP * EMBED, VCH).sum(axis=0)
         + t2[...].reshape(NCOMP * EMBED, VCH).sum(axis=0)
         + t3[...].reshape(NCOMP * EMBED, VCH).sum(axis=0)
         + mix[...].sum(axis=0))
    pos = i * VCH + lax.broadcasted_iota(jnp.int32, (VCH,), 0)
    out[...] = jnp.where(pos < VOCAB, s, 0.0)


def _gather_sum_body(widx_hbm, p_hbm, out_hbm, idx_v, g_v, outbuf, sem):
    wid = lax.axis_index("s") * NC + lax.axis_index("c")
    base = wid * BPW
    pltpu.sync_copy(widx_hbm.at[pl.ds(base, BPW)], idx_v)

    handles = [
        pltpu.async_copy(
            p_hbm.at[idx_v.at[pl.ds(ck * CHUNK, CHUNK)]],
            g_v.at[pl.ds(ck * CHUNK, CHUNK)],
            sem,
        )
        for ck in range(NCHUNK)
    ]
    for h in handles:
        h.wait()

    def body(i, m):
        return m + g_v[pl.ds(i * 16, 16)]

    outbuf[...] = lax.fori_loop(0, BPW // 16, body,
                                jnp.zeros((16,), jnp.float32))
    pltpu.sync_copy(outbuf, out_hbm.at[wid])


@jax.jit
def _run(word_idxs, tv0, tv1, tv2, tv3, mixv):
    p = pl.pallas_call(
        _plane_sum_body,
        grid=(NBLK,),
        in_specs=[
            pl.BlockSpec((NCOMP, EMBED, VCH), lambda i: (0, 0, i)),
            pl.BlockSpec((NCOMP, EMBED, VCH), lambda i: (0, 0, i)),
            pl.BlockSpec((NCOMP, EMBED, VCH), lambda i: (0, 0, i)),
            pl.BlockSpec((NCOMP, EMBED, VCH), lambda i: (0, 0, i)),
            pl.BlockSpec((NCOMP, VCH), lambda i: (0, i)),
        ],
        out_specs=pl.BlockSpec((VCH,), lambda i: (i,)),
        out_shape=jax.ShapeDtypeStruct((NBLK * VCH,), jnp.float32),
    )(tv0, tv1, tv2, tv3, mixv)

    mesh = plsc.VectorSubcoreMesh(core_axis_name="c", subcore_axis_name="s")
    f = pl.kernel(
        _gather_sum_body,
        out_type=jax.ShapeDtypeStruct((NW, 16), jnp.float32),
        mesh=mesh,
        scratch_types=[
            pltpu.VMEM((BPW,), jnp.int32),
            pltpu.VMEM((BPW,), jnp.float32),
            pltpu.VMEM((16,), jnp.float32),
            pltpu.SemaphoreType.DMA,
        ],
    )
    partials = f(word_idxs, p)
    return jnp.sum(partials)


def kernel(word_idxs, pos_idxs, neg_idxs, mus, logsigmas, mixture,
           mus_out, logsigmas_out):
    del pos_idxs, neg_idxs
    idx = word_idxs.astype(jnp.int32)
    tv0 = jnp.transpose(mus, (1, 2, 0))
    tv1 = jnp.transpose(logsigmas, (1, 2, 0))
    tv2 = jnp.transpose(mus_out, (1, 2, 0))
    tv3 = jnp.transpose(logsigmas_out, (1, 2, 0))
    mixv = jnp.transpose(mixture, (1, 0))
    return _run(idx, tv0, tv1, tv2, tv3, mixv)


# speedup vs baseline: 4.2726x; 1.0520x over previous
import jax
import jax.numpy as jnp
from jax import lax
from jax.experimental import pallas as pl
from jax.experimental.pallas import tpu as pltpu
from jax.experimental.pallas import tpu_sc as plsc

VOCAB = 100000
NCOMP = 2
EMBED = 64
B = 16384
NC = 2
NS = 16
NW = NC * NS
BPW = B // NW
CHUNK = 128
NCHUNK = BPW // CHUNK

VCH = 8192
NBLK = (VOCAB + VCH - 1) // VCH


def _plane_sum_body(t0, t1, t2, t3, mix, out):
    i = pl.program_id(0)
    s = (t0[...].reshape(NCOMP * EMBED, VCH).sum(axis=0)
         + t1[...].reshape(NCOMP * EMBED, VCH).sum(axis=0)
         + t2[...].reshape(NCOMP * EMBED, VCH).sum(axis=0)
         + t3[...].reshape(NCOMP * EMBED, VCH).sum(axis=0)
         + mix[...].sum(axis=0))
    pos = i * VCH + lax.broadcasted_iota(jnp.int32, (VCH,), 0)
    out[...] = jnp.where(pos < VOCAB, s, 0.0)


def _gather_sum_body(widx_hbm, p_hbm, out_hbm, idx_v, g_v, outbuf, sem):
    wid = lax.axis_index("s") * NC + lax.axis_index("c")
    base = wid * BPW
    pltpu.sync_copy(widx_hbm.at[pl.ds(base, BPW)], idx_v)

    handles = [
        pltpu.async_copy(
            p_hbm.at[idx_v.at[pl.ds(ck * CHUNK, CHUNK)]],
            g_v.at[pl.ds(ck * CHUNK, CHUNK)],
            sem,
        )
        for ck in range(NCHUNK)
    ]
    for h in handles:
        h.wait()

    def body(i, m):
        return m + g_v[pl.ds(i * 16, 16)]

    outbuf[...] = lax.fori_loop(0, BPW // 16, body,
                                jnp.zeros((16,), jnp.float32))
    pltpu.sync_copy(outbuf, out_hbm.at[wid])


@jax.jit
def _run(word_idxs, tv0, tv1, tv2, tv3, mixv):
    p = pl.pallas_call(
        _plane_sum_body,
        grid=(NBLK,),
        in_specs=[
            pl.BlockSpec((NCOMP, EMBED, VCH), lambda i: (0, 0, i)),
            pl.BlockSpec((NCOMP, EMBED, VCH), lambda i: (0, 0, i)),
            pl.BlockSpec((NCOMP, EMBED, VCH), lambda i: (0, 0, i)),
            pl.BlockSpec((NCOMP, EMBED, VCH), lambda i: (0, 0, i)),
            pl.BlockSpec((NCOMP, VCH), lambda i: (0, i)),
        ],
        out_specs=pl.BlockSpec((VCH,), lambda i: (i,)),
        out_shape=jax.ShapeDtypeStruct((NBLK * VCH,), jnp.float32),
    )(tv0, tv1, tv2, tv3, mixv)

    mesh = plsc.VectorSubcoreMesh(core_axis_name="c", subcore_axis_name="s")
    f = pl.kernel(
        _gather_sum_body,
        out_type=jax.ShapeDtypeStruct((NW, 16), jnp.float32),
        mesh=mesh,
        scratch_types=[
            pltpu.VMEM((BPW,), jnp.int32),
            pltpu.VMEM((BPW,), jnp.float32),
            pltpu.VMEM((16,), jnp.float32),
            pltpu.SemaphoreType.DMA,
        ],
    )
    partials = f(word_idxs, p)
    return jnp.sum(partials)


def kernel(word_idxs, pos_idxs, neg_idxs, mus, logsigmas, mixture,
           mus_out, logsigmas_out):
    del pos_idxs, neg_idxs
    idx = word_idxs.astype(jnp.int32)
    tv0 = jnp.transpose(mus, (1, 2, 0))
    tv1 = jnp.transpose(logsigmas, (1, 2, 0))
    tv2 = jnp.transpose(mus_out, (1, 2, 0))
    tv3 = jnp.transpose(logsigmas_out, (1, 2, 0))
    mixv = jnp.transpose(mixture, (1, 0))
    return _run(idx, tv0, tv1, tv2, tv3, mixv)
